# baseline (speedup 1.0000x reference)
import functools

import jax
import jax.numpy as jnp
from jax import lax
from jax.experimental import pallas as pl
from jax.experimental.pallas import tpu as pltpu
from jax.experimental.pallas import tpu_sc as plsc

N = 50000
E = 800000
NC, NS = 2, 16
NW = NC * NS
HALF = N // 2
EC = E // NW
CH = 128
NROW = E // CH
ROWS_PER_TILE = -(-NROW // NS)
TPT = 1568
SP_ROWS = TPT * NS
LAST_VALID = HALF - (NS - 1) * TPT

_MESH = plsc.VectorSubcoreMesh(
    core_axis_name="c", subcore_axis_name="s", num_cores=NC, num_subcores=NS)


def _prep_body(eflat, degflat, dstbuf, degtile):
    c = lax.axis_index("c")
    s = lax.axis_index("s")
    w = c * NS + s
    base = w * EC

    z16 = jnp.zeros((16,), jnp.float32)

    @pl.loop(0, N // 16)
    def _(i):
        degtile[pl.ds(i * 16, 16)] = z16

    pltpu.sync_copy(eflat.at[pl.ds(E + base, EC)], dstbuf)

    iota = lax.broadcasted_iota(jnp.int32, (16,), 0)
    nv = -(-EC // 16)

    @pl.loop(0, nv)
    def _(i):
        off = jnp.minimum(i * 16, EC - 16)
        v = dstbuf[pl.ds(off, 16)]
        val = jnp.where((off + iota) >= i * 16, 1.0, 0.0)
        plsc.addupdate_scatter(degtile, [v], val)

    pltpu.sync_copy(degtile, degflat.at[pl.ds(w * N, N)])


_prep = pl.kernel(
    _prep_body,
    out_type=jax.ShapeDtypeStruct((NW * N,), jnp.float32),
    mesh=_MESH,
    compiler_params=pltpu.CompilerParams(needs_layout_passes=False),
    scratch_types=[
        pltpu.VMEM((EC,), jnp.int32),
        pltpu.VMEM((N,), jnp.float32),
    ],
)


KB = 5
RB = NROW // NS
NBATCH = RB // KB
NEXTRA = NROW - NS * RB


NROW2 = NROW // 2
RB2 = NROW2 // NS
NBATCH2 = RB2 // KB
NEXTRA2 = NROW2 - NS * RB2
SP2_ROWS = 50048
TPT2 = SP2_ROWS // NS
LAST2 = N - (NS - 1) * TPT2


@functools.lru_cache(maxsize=None)
def _make_scatter_es(d):
    def body(hs, eflat, zrows, agg0, agg1, sidxb, didxb, rowsb, aggsp,
             gsem, ssem):
        c = lax.axis_index("c")
        s = lax.axis_index("s")

        pltpu.sync_copy(zrows, aggsp.at[pl.ds(s * TPT2, TPT2)])
        plsc.subcore_barrier()

        base = c * NROW2 + s * RB2

        @pl.loop(0, NBATCH2)
        def _(b):
            r0 = base + b * KB
            pltpu.sync_copy(eflat.at[pl.ds(r0 * CH, KB * CH)], sidxb)
            pltpu.sync_copy(eflat.at[pl.ds(E + r0 * CH, KB * CH)], didxb)
            pltpu.async_copy(hs.at[sidxb], rowsb, gsem).wait()
            pltpu.async_copy(rowsb, aggsp.at[didxb], ssem, add=True).wait()

        @pl.when(s < NEXTRA2)
        def _():
            row = c * NROW2 + NS * RB2 + s
            pltpu.sync_copy(eflat.at[pl.ds(row * CH, CH)],
                            sidxb.at[pl.ds(0, CH)])
            pltpu.sync_copy(eflat.at[pl.ds(E + row * CH, CH)],
                            didxb.at[pl.ds(0, CH)])
            pltpu.sync_copy(hs.at[sidxb.at[pl.ds(0, CH)]],
                            rowsb.at[pl.ds(0, CH)])
            pltpu.sync_copy(rowsb.at[pl.ds(0, CH)],
                            aggsp.at[didxb.at[pl.ds(0, CH)]], add=True)

        plsc.subcore_barrier()

        for cc, agg in ((0, agg0), (1, agg1)):
            @pl.when((c == cc) & (s < NS - 1))
            def _(agg=agg):
                pltpu.sync_copy(aggsp.at[pl.ds(s * TPT2, TPT2)],
                                agg.at[pl.ds(s * TPT2, TPT2)])

            @pl.when((c == cc) & (s == NS - 1))
            def _(agg=agg):
                pltpu.sync_copy(aggsp.at[pl.ds((NS - 1) * TPT2, LAST2)],
                                agg.at[pl.ds((NS - 1) * TPT2, LAST2)])

    return pl.kernel(
        body,
        out_type=(jax.ShapeDtypeStruct((N, d), jnp.float32),
                  jax.ShapeDtypeStruct((N, d), jnp.float32)),
        mesh=_MESH,
        compiler_params=pltpu.CompilerParams(
            needs_layout_passes=False, use_tc_tiling_on_sc=False),
        scratch_types=[
            pltpu.VMEM((KB * CH,), jnp.int32),
            pltpu.VMEM((KB * CH,), jnp.int32),
            pltpu.VMEM((KB * CH, d), jnp.float32),
            pltpu.VMEM_SHARED((SP2_ROWS, d), jnp.float32),
            pltpu.SemaphoreType.DMA,
            pltpu.SemaphoreType.DMA,
        ],
    )


@functools.lru_cache(maxsize=None)
def _make_scatter_fs(d):
    def body(hs2, eflat, zrows, agg0, agg1, sidxb, didxb, rowsb, aggsp,
             gsem, ssem):
        c = lax.axis_index("c")
        s = lax.axis_index("s")

        pltpu.sync_copy(zrows, aggsp.at[pl.ds(s * TPT2, TPT2)])
        plsc.subcore_barrier()

        base = s * RB

        @pl.loop(0, NBATCH)
        def _(b):
            r0 = base + b * KB
            pltpu.sync_copy(eflat.at[pl.ds(r0 * CH, KB * CH)], sidxb)
            pltpu.sync_copy(eflat.at[pl.ds(E + r0 * CH, KB * CH)], didxb)
            pltpu.async_copy(hs2.at[c].at[sidxb], rowsb, gsem).wait()
            pltpu.async_copy(rowsb, aggsp.at[didxb], ssem, add=True).wait()

        @pl.when(s < NEXTRA)
        def _():
            row = NS * RB + s
            pltpu.sync_copy(eflat.at[pl.ds(row * CH, CH)],
                            sidxb.at[pl.ds(0, CH)])
            pltpu.sync_copy(eflat.at[pl.ds(E + row * CH, CH)],
                            didxb.at[pl.ds(0, CH)])
            pltpu.sync_copy(hs2.at[c].at[sidxb.at[pl.ds(0, CH)]],
                            rowsb.at[pl.ds(0, CH)])
            pltpu.sync_copy(rowsb.at[pl.ds(0, CH)],
                            aggsp.at[didxb.at[pl.ds(0, CH)]], add=True)

        plsc.subcore_barrier()

        for cc, agg in ((0, agg0), (1, agg1)):
            @pl.when((c == cc) & (s < NS - 1))
            def _(agg=agg):
                pltpu.sync_copy(aggsp.at[pl.ds(s * TPT2, TPT2)],
                                agg.at[pl.ds(s * TPT2, TPT2)])

            @pl.when((c == cc) & (s == NS - 1))
            def _(agg=agg):
                pltpu.sync_copy(aggsp.at[pl.ds((NS - 1) * TPT2, LAST2)],
                                agg.at[pl.ds((NS - 1) * TPT2, LAST2)])

    return pl.kernel(
        body,
        out_type=(jax.ShapeDtypeStruct((N, d), jnp.float32),
                  jax.ShapeDtypeStruct((N, d), jnp.float32)),
        mesh=_MESH,
        compiler_params=pltpu.CompilerParams(
            needs_layout_passes=False, use_tc_tiling_on_sc=False),
        scratch_types=[
            pltpu.VMEM((KB * CH,), jnp.int32),
            pltpu.VMEM((KB * CH,), jnp.int32),
            pltpu.VMEM((KB * CH, d), jnp.float32),
            pltpu.VMEM_SHARED((SP2_ROWS, d), jnp.float32),
            pltpu.SemaphoreType.DMA,
            pltpu.SemaphoreType.DMA,
        ],
    )


_BM = 2000


def _mm1_body(x_ref, w_ref, o_ref):
    o_ref[...] = jnp.dot(x_ref[...], w_ref[...],
                         preferred_element_type=jnp.float32)


def _mm1(x, w):
    k = x.shape[1]
    d = w.shape[1]
    return pl.pallas_call(
        _mm1_body,
        grid=(N // _BM,),
        in_specs=[
            pl.BlockSpec((_BM, k), lambda i: (i, 0)),
            pl.BlockSpec((k, d), lambda i: (0, 0)),
        ],
        out_specs=pl.BlockSpec((_BM, d), lambda i: (i, 0)),
        out_shape=jax.ShapeDtypeStruct((N, d), jnp.float32),
    )(x, w)


_BS = 5000


def _scale_body(dpt_ref, h_ref, dis_ref, hs2_ref):
    deg = jnp.sum(dpt_ref[...], axis=1, keepdims=True) + 1.0
    dis = lax.rsqrt(deg)
    dis_ref[...] = dis
    hs = h_ref[...] * dis
    hs2_ref[0, ...] = hs[:, :32]
    hs2_ref[1, ...] = hs[:, 32:]


def _scale(dpt, h1):
    return pl.pallas_call(
        _scale_body,
        grid=(N // _BS,),
        in_specs=[
            pl.BlockSpec((_BS, NW), lambda i: (i, 0)),
            pl.BlockSpec((_BS, 64), lambda i: (i, 0)),
        ],
        out_specs=[
            pl.BlockSpec((_BS, 1), lambda i: (i, 0)),
            pl.BlockSpec((2, _BS, 32), lambda i: (0, i, 0)),
        ],
        out_shape=[
            jax.ShapeDtypeStruct((N, 1), jnp.float32),
            jax.ShapeDtypeStruct((2, N, 32), jnp.float32),
        ],
    )(dpt, h1)


def _fin1_body(aggA_ref, aggB_ref, hs2_ref, dis_ref, b_ref, w_ref, hso_ref):
    dis = dis_ref[...]
    za = dis * (aggA_ref[...] + hs2_ref[0, ...])
    zb = dis * (aggB_ref[...] + hs2_ref[1, ...])
    z = jnp.maximum(jnp.concatenate([za, zb], axis=1) + b_ref[...], 0.0)
    h = jnp.dot(z, w_ref[...], preferred_element_type=jnp.float32)
    hso_ref[...] = h * dis


def _fin1(aggA, aggB, hs2, dis, b, w):
    dout = w.shape[1]
    return pl.pallas_call(
        _fin1_body,
        grid=(N // _BS,),
        in_specs=[
            pl.BlockSpec((_BS, 32), lambda i: (i, 0)),
            pl.BlockSpec((_BS, 32), lambda i: (i, 0)),
            pl.BlockSpec((2, _BS, 32), lambda i: (0, i, 0)),
            pl.BlockSpec((_BS, 1), lambda i: (i, 0)),
            pl.BlockSpec((1, 64), lambda i: (0, 0)),
            pl.BlockSpec((64, dout), lambda i: (0, 0)),
        ],
        out_specs=pl.BlockSpec((_BS, dout), lambda i: (i, 0)),
        out_shape=jax.ShapeDtypeStruct((N, dout), jnp.float32),
    )(aggA, aggB, hs2, dis, b, w)


def _fin_body(agg_ref, hs_ref, dis_ref, b_ref, w_ref, hso_ref, agg1_ref=None):
    dis = dis_ref[...]
    a = agg_ref[...]
    if agg1_ref is not None:
        a = a + agg1_ref[...]
    z = jnp.maximum(dis * (a + hs_ref[...]) + b_ref[...], 0.0)
    h = jnp.dot(z, w_ref[...], preferred_element_type=jnp.float32)
    hso_ref[...] = h * dis


def _fin(agg, hs, dis, b, w, agg1=None):
    din = agg.shape[1]
    dout = w.shape[1]
    specs = [
        pl.BlockSpec((_BS, din), lambda i: (i, 0)),
        pl.BlockSpec((_BS, din), lambda i: (i, 0)),
        pl.BlockSpec((_BS, 1), lambda i: (i, 0)),
        pl.BlockSpec((1, din), lambda i: (0, 0)),
        pl.BlockSpec((din, dout), lambda i: (0, 0)),
    ]
    args = [agg, hs, dis, b, w]
    if agg1 is not None:
        specs.append(pl.BlockSpec((_BS, din), lambda i: (i, 0)))
        args.append(agg1)

        def bodyfn(a, h, di, bb, w_, a1, o):
            return _fin_body(a, h, di, bb, w_, o, a1)
    else:
        def bodyfn(a, h, di, bb, w_, o):
            return _fin_body(a, h, di, bb, w_, o)
    return pl.pallas_call(
        bodyfn,
        grid=(N // _BS,),
        in_specs=specs,
        out_specs=pl.BlockSpec((_BS, dout), lambda i: (i, 0)),
        out_shape=jax.ShapeDtypeStruct((N, dout), jnp.float32),
    )(*args)


def _last_body(agg_ref, agg1_ref, hs_ref, dis_ref, b_ref, o_ref):
    o_ref[...] = (dis_ref[...] * (agg_ref[...] + agg1_ref[...] + hs_ref[...])
                  + b_ref[...])


def _last(agg, agg1, hs, dis, b):
    d = agg.shape[1]
    return pl.pallas_call(
        _last_body,
        grid=(N // _BS,),
        in_specs=[
            pl.BlockSpec((_BS, d), lambda i: (i, 0)),
            pl.BlockSpec((_BS, d), lambda i: (i, 0)),
            pl.BlockSpec((_BS, d), lambda i: (i, 0)),
            pl.BlockSpec((_BS, 1), lambda i: (i, 0)),
            pl.BlockSpec((1, d), lambda i: (0, 0)),
        ],
        out_specs=pl.BlockSpec((_BS, d), lambda i: (i, 0)),
        out_shape=jax.ShapeDtypeStruct((N, d), jnp.float32),
    )(agg, agg1, hs, dis, b)


def _pad2(w, r, c):
    out = jnp.zeros((r, c), jnp.float32)
    return out.at[: w.shape[0], : w.shape[1]].set(w)


def _padb(b, c):
    return jnp.zeros((1, c), jnp.float32).at[0, : b.shape[0]].set(b)


def kernel(x, edge_index, y, W1, b1, W2, b2, W3, b3, W4, b4):
    del y
    d1, d2, d3, d4 = 64, 32, 16, 16

    eflat = jax.lax.optimization_barrier(edge_index.reshape(2 * E))
    W1p = _pad2(W1, 1433, d1)
    W2p = _pad2(W2, d1, d2)
    W3p = _pad2(W3, d2, d3)
    W4p = _pad2(W4, d3, d4)
    b1p, b2p, b3p = _padb(b1, d1), _padb(b2, d2), _padb(b3, d3)
    b4p = _padb(b4, d4)
    z2 = jnp.zeros((TPT2, d2), jnp.float32)
    z3 = jnp.zeros((TPT2, d3), jnp.float32)

    degflat = _prep(eflat)
    dpt = degflat.reshape(NW, N).T

    h1 = _mm1(x, W1p)
    dis, hs2x = _scale(dpt, h1)

    a1a, a1b = _make_scatter_fs(32)(hs2x, eflat, z2)
    hs2 = _fin1(a1a, a1b, hs2x, dis, b1p, W2p)
    a2a, a2b = _make_scatter_es(d2)(hs2, eflat, z2)
    hs3 = _fin(a2a, hs2, dis, b2p, W3p, agg1=a2b)
    a3a, a3b = _make_scatter_es(d3)(hs3, eflat, z3)
    hs4 = _fin(a3a, hs3, dis, b3p, W4p, agg1=a3b)
    a4a, a4b = _make_scatter_es(d3)(hs4, eflat, z3)
    out = _last(a4a, a4b, hs4, dis, b4p)
    return out[:, :7]


# speedup vs baseline: 1.0041x; 1.0041x over previous
import functools

import jax
import jax.numpy as jnp
from jax import lax
from jax.experimental import pallas as pl
from jax.experimental.pallas import tpu as pltpu
from jax.experimental.pallas import tpu_sc as plsc

N = 50000
E = 800000
NC, NS = 2, 16
NW = NC * NS
HALF = N // 2
EC = E // NW
CH = 128
NROW = E // CH
ROWS_PER_TILE = -(-NROW // NS)
TPT = 1568
SP_ROWS = TPT * NS
LAST_VALID = HALF - (NS - 1) * TPT

_MESH = plsc.VectorSubcoreMesh(
    core_axis_name="c", subcore_axis_name="s", num_cores=NC, num_subcores=NS)


def _prep_body(eflat, degflat, dstbuf, degtile):
    c = lax.axis_index("c")
    s = lax.axis_index("s")
    w = c * NS + s
    base = w * EC

    z16 = jnp.zeros((16,), jnp.float32)

    @pl.loop(0, N // 16)
    def _(i):
        degtile[pl.ds(i * 16, 16)] = z16

    pltpu.sync_copy(eflat.at[pl.ds(E + base, EC)], dstbuf)

    iota = lax.broadcasted_iota(jnp.int32, (16,), 0)
    nv = -(-EC // 16)

    @pl.loop(0, nv)
    def _(i):
        off = jnp.minimum(i * 16, EC - 16)
        v = dstbuf[pl.ds(off, 16)]
        val = jnp.where((off + iota) >= i * 16, 1.0, 0.0)
        plsc.addupdate_scatter(degtile, [v], val)

    pltpu.sync_copy(degtile, degflat.at[pl.ds(w * N, N)])


_prep = pl.kernel(
    _prep_body,
    out_type=jax.ShapeDtypeStruct((NW * N,), jnp.float32),
    mesh=_MESH,
    compiler_params=pltpu.CompilerParams(needs_layout_passes=False),
    scratch_types=[
        pltpu.VMEM((EC,), jnp.int32),
        pltpu.VMEM((N,), jnp.float32),
    ],
)


KB = 5
RB = NROW // NS
NBATCH = RB // KB
NEXTRA = NROW - NS * RB


NROW2 = NROW // 2
RB2 = NROW2 // NS
NBATCH2 = RB2 // KB
NEXTRA2 = NROW2 - NS * RB2
SP2_ROWS = 50048
TPT2 = SP2_ROWS // NS
LAST2 = N - (NS - 1) * TPT2


@functools.lru_cache(maxsize=None)
def _make_scatter_es(d):
    def body(hs, eflat, zrows, agg0, agg1, sidxb, didxb, rowsb, aggsp,
             gsem, ssem):
        c = lax.axis_index("c")
        s = lax.axis_index("s")

        pltpu.sync_copy(zrows, aggsp.at[pl.ds(s * TPT2, TPT2)])
        plsc.subcore_barrier()

        base = c * NROW2 + s * RB2

        @pl.loop(0, NBATCH2)
        def _(b):
            r0 = base + b * KB
            pltpu.sync_copy(eflat.at[pl.ds(r0 * CH, KB * CH)], sidxb)
            pltpu.sync_copy(eflat.at[pl.ds(E + r0 * CH, KB * CH)], didxb)
            pltpu.async_copy(hs.at[sidxb], rowsb, gsem).wait()
            pltpu.async_copy(rowsb, aggsp.at[didxb], ssem, add=True).wait()

        @pl.when(s < NEXTRA2)
        def _():
            row = c * NROW2 + NS * RB2 + s
            pltpu.sync_copy(eflat.at[pl.ds(row * CH, CH)],
                            sidxb.at[pl.ds(0, CH)])
            pltpu.sync_copy(eflat.at[pl.ds(E + row * CH, CH)],
                            didxb.at[pl.ds(0, CH)])
            pltpu.sync_copy(hs.at[sidxb.at[pl.ds(0, CH)]],
                            rowsb.at[pl.ds(0, CH)])
            pltpu.sync_copy(rowsb.at[pl.ds(0, CH)],
                            aggsp.at[didxb.at[pl.ds(0, CH)]], add=True)

        plsc.subcore_barrier()

        for cc, agg in ((0, agg0), (1, agg1)):
            @pl.when((c == cc) & (s < NS - 1))
            def _(agg=agg):
                pltpu.sync_copy(aggsp.at[pl.ds(s * TPT2, TPT2)],
                                agg.at[pl.ds(s * TPT2, TPT2)])

            @pl.when((c == cc) & (s == NS - 1))
            def _(agg=agg):
                pltpu.sync_copy(aggsp.at[pl.ds((NS - 1) * TPT2, LAST2)],
                                agg.at[pl.ds((NS - 1) * TPT2, LAST2)])

    return pl.kernel(
        body,
        out_type=(jax.ShapeDtypeStruct((N, d), jnp.float32),
                  jax.ShapeDtypeStruct((N, d), jnp.float32)),
        mesh=_MESH,
        compiler_params=pltpu.CompilerParams(
            needs_layout_passes=False, use_tc_tiling_on_sc=False),
        scratch_types=[
            pltpu.VMEM((KB * CH,), jnp.int32),
            pltpu.VMEM((KB * CH,), jnp.int32),
            pltpu.VMEM((KB * CH, d), jnp.float32),
            pltpu.VMEM_SHARED((SP2_ROWS, d), jnp.float32),
            pltpu.SemaphoreType.DMA,
            pltpu.SemaphoreType.DMA,
        ],
    )


@functools.lru_cache(maxsize=None)
def _make_scatter_fs(d):
    def body(hs2, eflat, zrows, agg0, agg1, sidxb, didxb, rowsb, aggsp,
             gsem, ssem):
        c = lax.axis_index("c")
        s = lax.axis_index("s")

        pltpu.sync_copy(zrows, aggsp.at[pl.ds(s * TPT2, TPT2)])
        plsc.subcore_barrier()

        base = s * RB

        @pl.loop(0, NBATCH)
        def _(b):
            r0 = base + b * KB
            pltpu.sync_copy(eflat.at[pl.ds(r0 * CH, KB * CH)], sidxb)
            pltpu.sync_copy(eflat.at[pl.ds(E + r0 * CH, KB * CH)], didxb)
            pltpu.async_copy(hs2.at[c].at[sidxb], rowsb, gsem).wait()
            pltpu.async_copy(rowsb, aggsp.at[didxb], ssem, add=True).wait()

        @pl.when(s < NEXTRA)
        def _():
            row = NS * RB + s
            pltpu.sync_copy(eflat.at[pl.ds(row * CH, CH)],
                            sidxb.at[pl.ds(0, CH)])
            pltpu.sync_copy(eflat.at[pl.ds(E + row * CH, CH)],
                            didxb.at[pl.ds(0, CH)])
            pltpu.sync_copy(hs2.at[c].at[sidxb.at[pl.ds(0, CH)]],
                            rowsb.at[pl.ds(0, CH)])
            pltpu.sync_copy(rowsb.at[pl.ds(0, CH)],
                            aggsp.at[didxb.at[pl.ds(0, CH)]], add=True)

        plsc.subcore_barrier()

        for cc, agg in ((0, agg0), (1, agg1)):
            @pl.when((c == cc) & (s < NS - 1))
            def _(agg=agg):
                pltpu.sync_copy(aggsp.at[pl.ds(s * TPT2, TPT2)],
                                agg.at[pl.ds(s * TPT2, TPT2)])

            @pl.when((c == cc) & (s == NS - 1))
            def _(agg=agg):
                pltpu.sync_copy(aggsp.at[pl.ds((NS - 1) * TPT2, LAST2)],
                                agg.at[pl.ds((NS - 1) * TPT2, LAST2)])

    return pl.kernel(
        body,
        out_type=(jax.ShapeDtypeStruct((N, d), jnp.float32),
                  jax.ShapeDtypeStruct((N, d), jnp.float32)),
        mesh=_MESH,
        compiler_params=pltpu.CompilerParams(
            needs_layout_passes=False, use_tc_tiling_on_sc=False),
        scratch_types=[
            pltpu.VMEM((KB * CH,), jnp.int32),
            pltpu.VMEM((KB * CH,), jnp.int32),
            pltpu.VMEM((KB * CH, d), jnp.float32),
            pltpu.VMEM_SHARED((SP2_ROWS, d), jnp.float32),
            pltpu.SemaphoreType.DMA,
            pltpu.SemaphoreType.DMA,
        ],
    )


_BM = 2000


def _mm1_body(x_ref, w_ref, o_ref):
    o_ref[...] = jnp.dot(x_ref[...], w_ref[...],
                         preferred_element_type=jnp.float32)


def _mm1(x, w):
    k = x.shape[1]
    d = w.shape[1]
    return pl.pallas_call(
        _mm1_body,
        grid=(N // _BM,),
        in_specs=[
            pl.BlockSpec((_BM, k), lambda i: (i, 0)),
            pl.BlockSpec((k, d), lambda i: (0, 0)),
        ],
        out_specs=pl.BlockSpec((_BM, d), lambda i: (i, 0)),
        out_shape=jax.ShapeDtypeStruct((N, d), jnp.float32),
    )(x, w)


def _eflat_body(x_ref, o_ref):
    o_ref[pl.ds(0, E)] = x_ref[0, :]
    o_ref[pl.ds(E, E)] = x_ref[1, :]


def _eflat_copy(edge_index):
    return pl.pallas_call(
        _eflat_body,
        out_shape=jax.ShapeDtypeStruct((2 * E,), jnp.int32),
    )(edge_index)


_BS = 5000


def _scale_body(dpt_ref, h_ref, dis_ref, hs2_ref):
    deg = jnp.sum(dpt_ref[...], axis=1, keepdims=True) + 1.0
    dis = lax.rsqrt(deg)
    dis_ref[...] = dis
    hs = h_ref[...] * dis
    hs2_ref[0, ...] = hs[:, :32]
    hs2_ref[1, ...] = hs[:, 32:]


def _scale(dpt, h1):
    return pl.pallas_call(
        _scale_body,
        grid=(N // _BS,),
        in_specs=[
            pl.BlockSpec((_BS, NW), lambda i: (i, 0)),
            pl.BlockSpec((_BS, 64), lambda i: (i, 0)),
        ],
        out_specs=[
            pl.BlockSpec((_BS, 1), lambda i: (i, 0)),
            pl.BlockSpec((2, _BS, 32), lambda i: (0, i, 0)),
        ],
        out_shape=[
            jax.ShapeDtypeStruct((N, 1), jnp.float32),
            jax.ShapeDtypeStruct((2, N, 32), jnp.float32),
        ],
    )(dpt, h1)


def _fin1_body(aggA_ref, aggB_ref, hs2_ref, dis_ref, b_ref, w_ref, hso_ref):
    dis = dis_ref[...]
    za = dis * (aggA_ref[...] + hs2_ref[0, ...])
    zb = dis * (aggB_ref[...] + hs2_ref[1, ...])
    z = jnp.maximum(jnp.concatenate([za, zb], axis=1) + b_ref[...], 0.0)
    h = jnp.dot(z, w_ref[...], preferred_element_type=jnp.float32)
    hso_ref[...] = h * dis


def _fin1(aggA, aggB, hs2, dis, b, w):
    dout = w.shape[1]
    return pl.pallas_call(
        _fin1_body,
        grid=(N // _BS,),
        in_specs=[
            pl.BlockSpec((_BS, 32), lambda i: (i, 0)),
            pl.BlockSpec((_BS, 32), lambda i: (i, 0)),
            pl.BlockSpec((2, _BS, 32), lambda i: (0, i, 0)),
            pl.BlockSpec((_BS, 1), lambda i: (i, 0)),
            pl.BlockSpec((1, 64), lambda i: (0, 0)),
            pl.BlockSpec((64, dout), lambda i: (0, 0)),
        ],
        out_specs=pl.BlockSpec((_BS, dout), lambda i: (i, 0)),
        out_shape=jax.ShapeDtypeStruct((N, dout), jnp.float32),
    )(aggA, aggB, hs2, dis, b, w)


def _fin_body(agg_ref, hs_ref, dis_ref, b_ref, w_ref, hso_ref, agg1_ref=None):
    dis = dis_ref[...]
    a = agg_ref[...]
    if agg1_ref is not None:
        a = a + agg1_ref[...]
    z = jnp.maximum(dis * (a + hs_ref[...]) + b_ref[...], 0.0)
    h = jnp.dot(z, w_ref[...], preferred_element_type=jnp.float32)
    hso_ref[...] = h * dis


def _fin(agg, hs, dis, b, w, agg1=None):
    din = agg.shape[1]
    dout = w.shape[1]
    specs = [
        pl.BlockSpec((_BS, din), lambda i: (i, 0)),
        pl.BlockSpec((_BS, din), lambda i: (i, 0)),
        pl.BlockSpec((_BS, 1), lambda i: (i, 0)),
        pl.BlockSpec((1, din), lambda i: (0, 0)),
        pl.BlockSpec((din, dout), lambda i: (0, 0)),
    ]
    args = [agg, hs, dis, b, w]
    if agg1 is not None:
        specs.append(pl.BlockSpec((_BS, din), lambda i: (i, 0)))
        args.append(agg1)

        def bodyfn(a, h, di, bb, w_, a1, o):
            return _fin_body(a, h, di, bb, w_, o, a1)
    else:
        def bodyfn(a, h, di, bb, w_, o):
            return _fin_body(a, h, di, bb, w_, o)
    return pl.pallas_call(
        bodyfn,
        grid=(N // _BS,),
        in_specs=specs,
        out_specs=pl.BlockSpec((_BS, dout), lambda i: (i, 0)),
        out_shape=jax.ShapeDtypeStruct((N, dout), jnp.float32),
    )(*args)


def _last_body(agg_ref, agg1_ref, hs_ref, dis_ref, b_ref, o_ref):
    o_ref[...] = (dis_ref[...] * (agg_ref[...] + agg1_ref[...] + hs_ref[...])
                  + b_ref[...])


def _last(agg, agg1, hs, dis, b):
    d = agg.shape[1]
    return pl.pallas_call(
        _last_body,
        grid=(N // _BS,),
        in_specs=[
            pl.BlockSpec((_BS, d), lambda i: (i, 0)),
            pl.BlockSpec((_BS, d), lambda i: (i, 0)),
            pl.BlockSpec((_BS, d), lambda i: (i, 0)),
            pl.BlockSpec((_BS, 1), lambda i: (i, 0)),
            pl.BlockSpec((1, d), lambda i: (0, 0)),
        ],
        out_specs=pl.BlockSpec((_BS, d), lambda i: (i, 0)),
        out_shape=jax.ShapeDtypeStruct((N, d), jnp.float32),
    )(agg, agg1, hs, dis, b)


def _pad2(w, r, c):
    out = jnp.zeros((r, c), jnp.float32)
    return out.at[: w.shape[0], : w.shape[1]].set(w)


def _padb(b, c):
    return jnp.zeros((1, c), jnp.float32).at[0, : b.shape[0]].set(b)


def kernel(x, edge_index, y, W1, b1, W2, b2, W3, b3, W4, b4):
    del y
    d1, d2, d3, d4 = 64, 32, 16, 16

    eflat = _eflat_copy(edge_index)
    W1p = _pad2(W1, 1433, d1)
    W2p = _pad2(W2, d1, d2)
    W3p = _pad2(W3, d2, d3)
    W4p = _pad2(W4, d3, d4)
    b1p, b2p, b3p = _padb(b1, d1), _padb(b2, d2), _padb(b3, d3)
    b4p = _padb(b4, d4)
    z2 = jnp.zeros((TPT2, d2), jnp.float32)
    z3 = jnp.zeros((TPT2, d3), jnp.float32)

    degflat = _prep(eflat)
    dpt = degflat.reshape(NW, N).T

    h1 = _mm1(x, W1p)
    dis, hs2x = _scale(dpt, h1)

    a1a, a1b = _make_scatter_fs(32)(hs2x, eflat, z2)
    hs2 = _fin1(a1a, a1b, hs2x, dis, b1p, W2p)
    a2a, a2b = _make_scatter_es(d2)(hs2, eflat, z2)
    hs3 = _fin(a2a, hs2, dis, b2p, W3p, agg1=a2b)
    a3a, a3b = _make_scatter_es(d3)(hs3, eflat, z3)
    hs4 = _fin(a3a, hs3, dis, b3p, W4p, agg1=a3b)
    a4a, a4b = _make_scatter_es(d3)(hs4, eflat, z3)
    out = _last(a4a, a4b, hs4, dis, b4p)
    return out[:, :7]


# speedup vs baseline: 1.2012x; 1.1963x over previous
import functools

import jax
import jax.numpy as jnp
from jax import lax
from jax.experimental import pallas as pl
from jax.experimental.pallas import tpu as pltpu
from jax.experimental.pallas import tpu_sc as plsc

N = 50000
E = 800000
NC, NS = 2, 16
NW = NC * NS
HALF = N // 2
EC = E // NW
CH = 128
NROW = E // CH
ROWS_PER_TILE = -(-NROW // NS)
TPT = 1568
SP_ROWS = TPT * NS
LAST_VALID = HALF - (NS - 1) * TPT

_MESH = plsc.VectorSubcoreMesh(
    core_axis_name="c", subcore_axis_name="s", num_cores=NC, num_subcores=NS)


def _prep_body(eflat, degflat, dstbuf, degtile):
    c = lax.axis_index("c")
    s = lax.axis_index("s")
    w = c * NS + s
    base = w * EC

    z16 = jnp.zeros((16,), jnp.float32)

    @pl.loop(0, N // 16)
    def _(i):
        degtile[pl.ds(i * 16, 16)] = z16

    pltpu.sync_copy(eflat.at[pl.ds(E + base, EC)], dstbuf)

    iota = lax.broadcasted_iota(jnp.int32, (16,), 0)
    nv = -(-EC // 16)

    @pl.loop(0, nv)
    def _(i):
        off = jnp.minimum(i * 16, EC - 16)
        v = dstbuf[pl.ds(off, 16)]
        val = jnp.where((off + iota) >= i * 16, 1.0, 0.0)
        plsc.addupdate_scatter(degtile, [v], val)

    pltpu.sync_copy(degtile, degflat.at[pl.ds(w * N, N)])


_prep = pl.kernel(
    _prep_body,
    out_type=jax.ShapeDtypeStruct((NW * N,), jnp.float32),
    mesh=_MESH,
    compiler_params=pltpu.CompilerParams(needs_layout_passes=False),
    scratch_types=[
        pltpu.VMEM((EC,), jnp.int32),
        pltpu.VMEM((N,), jnp.float32),
    ],
)


KB = 5
RB = NROW // NS
NBATCH = RB // KB
NEXTRA = NROW - NS * RB


NROW2 = NROW // 2
RB2 = NROW2 // NS
NBATCH2 = RB2 // KB
NEXTRA2 = NROW2 - NS * RB2
SP2_ROWS = 50048
TPT2 = SP2_ROWS // NS
LAST2 = N - (NS - 1) * TPT2


@functools.lru_cache(maxsize=None)
def _make_scatter_es(d):
    def body(hs, eflat, zrows, agg0, agg1, sidxb, didxb, rowsb, aggsp,
             gsem, ssem):
        c = lax.axis_index("c")
        s = lax.axis_index("s")

        pltpu.sync_copy(zrows, aggsp.at[pl.ds(s * TPT2, TPT2)])
        plsc.subcore_barrier()

        base = c * NROW2 + s * RB2

        @pl.loop(0, NBATCH2)
        def _(b):
            r0 = base + b * KB
            pltpu.sync_copy(eflat.at[pl.ds(r0 * CH, KB * CH)], sidxb)
            pltpu.sync_copy(eflat.at[pl.ds(E + r0 * CH, KB * CH)], didxb)
            pltpu.async_copy(hs.at[sidxb], rowsb, gsem).wait()
            pltpu.async_copy(rowsb, aggsp.at[didxb], ssem, add=True).wait()

        @pl.when(s < NEXTRA2)
        def _():
            row = c * NROW2 + NS * RB2 + s
            pltpu.sync_copy(eflat.at[pl.ds(row * CH, CH)],
                            sidxb.at[pl.ds(0, CH)])
            pltpu.sync_copy(eflat.at[pl.ds(E + row * CH, CH)],
                            didxb.at[pl.ds(0, CH)])
            pltpu.sync_copy(hs.at[sidxb.at[pl.ds(0, CH)]],
                            rowsb.at[pl.ds(0, CH)])
            pltpu.sync_copy(rowsb.at[pl.ds(0, CH)],
                            aggsp.at[didxb.at[pl.ds(0, CH)]], add=True)

        plsc.subcore_barrier()

        for cc, agg in ((0, agg0), (1, agg1)):
            @pl.when((c == cc) & (s < NS - 1))
            def _(agg=agg):
                pltpu.sync_copy(aggsp.at[pl.ds(s * TPT2, TPT2)],
                                agg.at[pl.ds(s * TPT2, TPT2)])

            @pl.when((c == cc) & (s == NS - 1))
            def _(agg=agg):
                pltpu.sync_copy(aggsp.at[pl.ds((NS - 1) * TPT2, LAST2)],
                                agg.at[pl.ds((NS - 1) * TPT2, LAST2)])

    return pl.kernel(
        body,
        out_type=(jax.ShapeDtypeStruct((N, d), jnp.float32),
                  jax.ShapeDtypeStruct((N, d), jnp.float32)),
        mesh=_MESH,
        compiler_params=pltpu.CompilerParams(
            needs_layout_passes=False, use_tc_tiling_on_sc=False),
        scratch_types=[
            pltpu.VMEM((KB * CH,), jnp.int32),
            pltpu.VMEM((KB * CH,), jnp.int32),
            pltpu.VMEM((KB * CH, d), jnp.float32),
            pltpu.VMEM_SHARED((SP2_ROWS, d), jnp.float32),
            pltpu.SemaphoreType.DMA,
            pltpu.SemaphoreType.DMA,
        ],
    )


@functools.lru_cache(maxsize=None)
def _make_scatter_fs(d):
    def body(hs2, eflat, zrows, agg0, agg1, sidxb, didxb, rowsb, aggsp,
             gsem, ssem):
        c = lax.axis_index("c")
        s = lax.axis_index("s")

        pltpu.sync_copy(zrows, aggsp.at[pl.ds(s * TPT2, TPT2)])
        plsc.subcore_barrier()

        base = s * RB

        @pl.loop(0, NBATCH)
        def _(b):
            r0 = base + b * KB
            pltpu.sync_copy(eflat.at[pl.ds(r0 * CH, KB * CH)], sidxb)
            pltpu.sync_copy(eflat.at[pl.ds(E + r0 * CH, KB * CH)], didxb)
            pltpu.async_copy(hs2.at[c].at[sidxb], rowsb, gsem).wait()
            pltpu.async_copy(rowsb, aggsp.at[didxb], ssem, add=True).wait()

        @pl.when(s < NEXTRA)
        def _():
            row = NS * RB + s
            pltpu.sync_copy(eflat.at[pl.ds(row * CH, CH)],
                            sidxb.at[pl.ds(0, CH)])
            pltpu.sync_copy(eflat.at[pl.ds(E + row * CH, CH)],
                            didxb.at[pl.ds(0, CH)])
            pltpu.sync_copy(hs2.at[c].at[sidxb.at[pl.ds(0, CH)]],
                            rowsb.at[pl.ds(0, CH)])
            pltpu.sync_copy(rowsb.at[pl.ds(0, CH)],
                            aggsp.at[didxb.at[pl.ds(0, CH)]], add=True)

        plsc.subcore_barrier()

        for cc, agg in ((0, agg0), (1, agg1)):
            @pl.when((c == cc) & (s < NS - 1))
            def _(agg=agg):
                pltpu.sync_copy(aggsp.at[pl.ds(s * TPT2, TPT2)],
                                agg.at[pl.ds(s * TPT2, TPT2)])

            @pl.when((c == cc) & (s == NS - 1))
            def _(agg=agg):
                pltpu.sync_copy(aggsp.at[pl.ds((NS - 1) * TPT2, LAST2)],
                                agg.at[pl.ds((NS - 1) * TPT2, LAST2)])

    return pl.kernel(
        body,
        out_type=(jax.ShapeDtypeStruct((N, d), jnp.float32),
                  jax.ShapeDtypeStruct((N, d), jnp.float32)),
        mesh=_MESH,
        compiler_params=pltpu.CompilerParams(
            needs_layout_passes=False, use_tc_tiling_on_sc=False),
        scratch_types=[
            pltpu.VMEM((KB * CH,), jnp.int32),
            pltpu.VMEM((KB * CH,), jnp.int32),
            pltpu.VMEM((KB * CH, d), jnp.float32),
            pltpu.VMEM_SHARED((SP2_ROWS, d), jnp.float32),
            pltpu.SemaphoreType.DMA,
            pltpu.SemaphoreType.DMA,
        ],
    )


_BM = 2048


def _mm1_body(xt_ref, w_ref, o_ref):
    o_ref[...] = lax.dot_general(
        xt_ref[...], w_ref[...],
        dimension_numbers=(((0,), (0,)), ((), ())),
        preferred_element_type=jnp.float32)


def _mm1(xt, w):
    k = xt.shape[0]
    d = w.shape[1]
    return pl.pallas_call(
        _mm1_body,
        grid=(-(-N // _BM),),
        in_specs=[
            pl.BlockSpec((k, _BM), lambda i: (0, i)),
            pl.BlockSpec((k, d), lambda i: (0, 0)),
        ],
        out_specs=pl.BlockSpec((_BM, d), lambda i: (i, 0)),
        out_shape=jax.ShapeDtypeStruct((N, d), jnp.float32),
    )(xt, w)


def _eflat_body(x_ref, o_ref):
    o_ref[pl.ds(0, E)] = x_ref[0, :]
    o_ref[pl.ds(E, E)] = x_ref[1, :]


def _eflat_copy(edge_index):
    return pl.pallas_call(
        _eflat_body,
        out_shape=jax.ShapeDtypeStruct((2 * E,), jnp.int32),
    )(edge_index)


_BS = 5000


def _deg_body(dp_ref, dis_ref):
    deg = jnp.sum(dp_ref[...], axis=0, keepdims=True) + 1.0
    dis_ref[...] = lax.rsqrt(deg).reshape(N, 1)


def _deg(dp):
    return pl.pallas_call(
        _deg_body,
        in_specs=[pl.BlockSpec((NW, N), lambda: (0, 0))],
        out_specs=pl.BlockSpec((N, 1), lambda: (0, 0)),
        out_shape=jax.ShapeDtypeStruct((N, 1), jnp.float32),
    )(dp)


def _scale_body(dis_ref, h_ref, hs2_ref):
    hs = h_ref[...] * dis_ref[...]
    hs2_ref[0, ...] = hs[:, :32]
    hs2_ref[1, ...] = hs[:, 32:]


def _scale(dis, h1):
    return pl.pallas_call(
        _scale_body,
        grid=(N // _BS,),
        in_specs=[
            pl.BlockSpec((_BS, 1), lambda i: (i, 0)),
            pl.BlockSpec((_BS, 64), lambda i: (i, 0)),
        ],
        out_specs=pl.BlockSpec((2, _BS, 32), lambda i: (0, i, 0)),
        out_shape=jax.ShapeDtypeStruct((2, N, 32), jnp.float32),
    )(dis, h1)


def _fin1_body(aggA_ref, aggB_ref, hs2_ref, dis_ref, b_ref, w_ref, hso_ref):
    dis = dis_ref[...]
    za = dis * (aggA_ref[...] + hs2_ref[0, ...])
    zb = dis * (aggB_ref[...] + hs2_ref[1, ...])
    z = jnp.maximum(jnp.concatenate([za, zb], axis=1) + b_ref[...], 0.0)
    h = jnp.dot(z, w_ref[...], preferred_element_type=jnp.float32)
    hso_ref[...] = h * dis


def _fin1(aggA, aggB, hs2, dis, b, w):
    dout = w.shape[1]
    return pl.pallas_call(
        _fin1_body,
        grid=(N // _BS,),
        in_specs=[
            pl.BlockSpec((_BS, 32), lambda i: (i, 0)),
            pl.BlockSpec((_BS, 32), lambda i: (i, 0)),
            pl.BlockSpec((2, _BS, 32), lambda i: (0, i, 0)),
            pl.BlockSpec((_BS, 1), lambda i: (i, 0)),
            pl.BlockSpec((1, 64), lambda i: (0, 0)),
            pl.BlockSpec((64, dout), lambda i: (0, 0)),
        ],
        out_specs=pl.BlockSpec((_BS, dout), lambda i: (i, 0)),
        out_shape=jax.ShapeDtypeStruct((N, dout), jnp.float32),
    )(aggA, aggB, hs2, dis, b, w)


def _fin_body(agg_ref, hs_ref, dis_ref, b_ref, w_ref, hso_ref, agg1_ref=None):
    dis = dis_ref[...]
    a = agg_ref[...]
    if agg1_ref is not None:
        a = a + agg1_ref[...]
    z = jnp.maximum(dis * (a + hs_ref[...]) + b_ref[...], 0.0)
    h = jnp.dot(z, w_ref[...], preferred_element_type=jnp.float32)
    hso_ref[...] = h * dis


def _fin(agg, hs, dis, b, w, agg1=None):
    din = agg.shape[1]
    dout = w.shape[1]
    specs = [
        pl.BlockSpec((_BS, din), lambda i: (i, 0)),
        pl.BlockSpec((_BS, din), lambda i: (i, 0)),
        pl.BlockSpec((_BS, 1), lambda i: (i, 0)),
        pl.BlockSpec((1, din), lambda i: (0, 0)),
        pl.BlockSpec((din, dout), lambda i: (0, 0)),
    ]
    args = [agg, hs, dis, b, w]
    if agg1 is not None:
        specs.append(pl.BlockSpec((_BS, din), lambda i: (i, 0)))
        args.append(agg1)

        def bodyfn(a, h, di, bb, w_, a1, o):
            return _fin_body(a, h, di, bb, w_, o, a1)
    else:
        def bodyfn(a, h, di, bb, w_, o):
            return _fin_body(a, h, di, bb, w_, o)
    return pl.pallas_call(
        bodyfn,
        grid=(N // _BS,),
        in_specs=specs,
        out_specs=pl.BlockSpec((_BS, dout), lambda i: (i, 0)),
        out_shape=jax.ShapeDtypeStruct((N, dout), jnp.float32),
    )(*args)


def _last_body(agg_ref, agg1_ref, hs_ref, dis_ref, b_ref, o_ref):
    o_ref[...] = (dis_ref[...] * (agg_ref[...] + agg1_ref[...] + hs_ref[...])
                  + b_ref[...])


def _last(agg, agg1, hs, dis, b):
    d = agg.shape[1]
    return pl.pallas_call(
        _last_body,
        grid=(N // _BS,),
        in_specs=[
            pl.BlockSpec((_BS, d), lambda i: (i, 0)),
            pl.BlockSpec((_BS, d), lambda i: (i, 0)),
            pl.BlockSpec((_BS, d), lambda i: (i, 0)),
            pl.BlockSpec((_BS, 1), lambda i: (i, 0)),
            pl.BlockSpec((1, d), lambda i: (0, 0)),
        ],
        out_specs=pl.BlockSpec((_BS, d), lambda i: (i, 0)),
        out_shape=jax.ShapeDtypeStruct((N, d), jnp.float32),
    )(agg, agg1, hs, dis, b)


def _pad2(w, r, c):
    out = jnp.zeros((r, c), jnp.float32)
    return out.at[: w.shape[0], : w.shape[1]].set(w)


def _padb(b, c):
    return jnp.zeros((1, c), jnp.float32).at[0, : b.shape[0]].set(b)


def kernel(x, edge_index, y, W1, b1, W2, b2, W3, b3, W4, b4):
    del y
    d1, d2, d3, d4 = 64, 32, 16, 16

    eflat = _eflat_copy(edge_index)
    W1p = _pad2(W1, 1433, d1)
    W2p = _pad2(W2, d1, d2)
    W3p = _pad2(W3, d2, d3)
    W4p = _pad2(W4, d3, d4)
    b1p, b2p, b3p = _padb(b1, d1), _padb(b2, d2), _padb(b3, d3)
    b4p = _padb(b4, d4)
    z2 = jnp.zeros((TPT2, d2), jnp.float32)
    z3 = jnp.zeros((TPT2, d3), jnp.float32)

    degflat = _prep(eflat)

    h1 = _mm1(x.T, W1p)
    dis = _deg(degflat.reshape(NW, N))
    hs2x = _scale(dis, h1)

    a1a, a1b = _make_scatter_fs(32)(hs2x, eflat, z2)
    hs2 = _fin1(a1a, a1b, hs2x, dis, b1p, W2p)
    a2a, a2b = _make_scatter_es(d2)(hs2, eflat, z2)
    hs3 = _fin(a2a, hs2, dis, b2p, W3p, agg1=a2b)
    a3a, a3b = _make_scatter_es(d3)(hs3, eflat, z3)
    hs4 = _fin(a3a, hs3, dis, b3p, W4p, agg1=a3b)
    a4a, a4b = _make_scatter_es(d3)(hs4, eflat, z3)
    out = _last(a4a, a4b, hs4, dis, b4p)
    return out[:, :7]
